# baseline (speedup 1.0000x reference)
import functools

import jax
import jax.numpy as jnp
from jax import lax
from jax.experimental import pallas as pl
from jax.experimental.pallas import tpu as pltpu
from jax.experimental.pallas import tpu_sc as plsc

NC = 2
NS = 16
L = 16
NW = NC * NS

B = 32
H = 512
W = 512
N = B * H * W
NBINS = 5
U = 8
RB = 128

ONE_BITS = 0x3F800000

K_SC = 4

_MESH = dict(core_axis_name="c", subcore_axis_name="s")


def _sc_fast(pred, targ, k):
    rows_w = k * H // NW
    r_chunk = min(32, rows_w)
    nchunk = rows_w // r_chunk
    blocks = r_chunk * (W // (L * U))

    @functools.partial(
        pl.kernel,
        out_type=jax.ShapeDtypeStruct((NW * 2 * L,), jnp.float32),
        mesh=plsc.VectorSubcoreMesh(**_MESH),
        scratch_types=[
            pltpu.VMEM((r_chunk, W), jnp.float32),
            pltpu.VMEM((r_chunk, W), jnp.float32),
            pltpu.VMEM((r_chunk, W), jnp.float32),
            pltpu.VMEM((r_chunk, W), jnp.float32),
            pltpu.VMEM((2 * L,), jnp.float32),
            pltpu.SemaphoreType.DMA,
            pltpu.SemaphoreType.DMA,
        ],
    )
    def body(pred_hbm, targ_hbm, out_hbm, pb0, tb0, pb1, tb1, ov, sm0, sm1):
        wid = lax.axis_index("s") * NC + lax.axis_index("c")
        r_abs = wid * rows_w
        img = lax.shift_right_logical(r_abs, 9)
        r_base = lax.bitwise_and(r_abs, H - 1)
        bufs = [(pb0, tb0, sm0), (pb1, tb1, sm1)]

        def start(g):
            pb, tb, sm = bufs[g % 2]
            r0 = pl.multiple_of(r_base + g * r_chunk, r_chunk)
            hp = pltpu.async_copy(pred_hbm.at[img, pl.ds(r0, r_chunk), :],
                                  pb, sm)
            ht = pltpu.async_copy(targ_hbm.at[img, pl.ds(r0, r_chunk), :],
                                  tb, sm)
            return (hp, ht)

        zero = jnp.zeros((L,), jnp.float32)
        uzero = jnp.zeros((L,), jnp.uint32)

        def make_it(pb, tb):
            def it(i, c):
                accs = list(c[:U])
                mx0, mx1 = c[U], c[U + 1]
                row = lax.shift_right_logical(i, 2)
                cb = pl.multiple_of(
                    lax.shift_left(lax.bitwise_and(i, 3), 7), 128)
                for j in range(U):
                    p = pb[row, pl.ds(cb + j * L, L)]
                    t = tb[row, pl.ds(cb + j * L, L)]
                    d = p - t
                    accs[j] = accs[j] + d * d
                    bu = lax.bitcast_convert_type(t, jnp.uint32)
                    if j % 2 == 0:
                        mx0 = jnp.where(bu > mx0, bu, mx0)
                    else:
                        mx1 = jnp.where(bu > mx1, bu, mx1)
                return (*accs, mx0, mx1)
            return it

        pending = {0: start(0)}
        carry = tuple(zero for _ in range(U)) + (uzero, uzero)
        for g in range(nchunk):
            if g + 1 < nchunk:
                pending[g + 1] = start(g + 1)
            for h in pending.pop(g):
                h.wait()
            pb, tb, _ = bufs[g % 2]
            carry = lax.fori_loop(0, blocks, make_it(pb, tb), carry)

        acc = carry[0]
        for j in range(1, U):
            acc = acc + carry[j]
        mx0, mx1 = carry[U], carry[U + 1]
        mx = jnp.where(mx0 > mx1, mx0, mx1)
        ov[pl.ds(0, L)] = acc
        ov[pl.ds(L, L)] = lax.bitcast_convert_type(mx, jnp.float32)
        pltpu.sync_copy(ov, out_hbm.at[pl.ds(wid * 2 * L, 2 * L)])

    return body(pred, targ)


def _tc_fast(pred, targ, k0, nimg):
    def body(p_ref, t_ref, s_ref, mx_ref, mn_ref):
        i = pl.program_id(0)
        j = pl.program_id(1)

        @pl.when(jnp.logical_and(i == 0, j == 0))
        def _():
            s_ref[...] = jnp.zeros_like(s_ref)
            mx_ref[...] = jnp.full_like(mx_ref, jnp.iinfo(jnp.int32).min)
            mn_ref[...] = jnp.full_like(mn_ref, jnp.iinfo(jnp.int32).max)

        p = p_ref[0]
        t = t_ref[0]
        d = p - t
        d2 = d * d
        bi = lax.bitcast_convert_type(t, jnp.int32)
        for c in range(W // 128):
            cs = slice(c * 128, (c + 1) * 128)
            part = s_ref[:, cs]
            mx = mx_ref[:, cs]
            mn = mn_ref[:, cs]
            for r in range(RB // 8):
                rs = slice(r * 8, (r + 1) * 8)
                part = part + d2[rs, cs]
                tile = bi[rs, cs]
                mx = jnp.maximum(mx, tile)
                mn = jnp.minimum(mn, tile)
            s_ref[:, cs] = part
            mx_ref[:, cs] = mx
            mn_ref[:, cs] = mn

    s, mx, mn = pl.pallas_call(
        body,
        grid=(nimg, H // RB),
        in_specs=[
            pl.BlockSpec((1, RB, W), lambda i, j: (i + k0, j, 0)),
            pl.BlockSpec((1, RB, W), lambda i, j: (i + k0, j, 0)),
        ],
        out_specs=[
            pl.BlockSpec((8, W), lambda i, j: (0, 0)),
            pl.BlockSpec((8, W), lambda i, j: (0, 0)),
            pl.BlockSpec((8, W), lambda i, j: (0, 0)),
        ],
        out_shape=[
            jax.ShapeDtypeStruct((8, W), jnp.float32),
            jax.ShapeDtypeStruct((8, W), jnp.int32),
            jax.ShapeDtypeStruct((8, W), jnp.int32),
        ],
    )(pred, targ)
    return s, mx, mn


def _sc_general(pred, targ):
    R = 32
    nchunk = H // R

    @functools.partial(
        pl.kernel,
        out_type=jax.ShapeDtypeStruct((NW * NBINS * L,), jnp.float32),
        mesh=plsc.VectorSubcoreMesh(**_MESH),
        scratch_types=[
            pltpu.VMEM((R, W), jnp.float32),
            pltpu.VMEM((R, W), jnp.float32),
            pltpu.VMEM((NBINS * L,), jnp.float32),
        ],
    )
    def body(pred_hbm, targ_hbm, out_hbm, pbuf, tbuf, ov):
        wid = lax.axis_index("s") * NC + lax.axis_index("c")

        zero = jnp.zeros((L,), jnp.float32)
        accs = tuple(zero for _ in range(NBINS))
        for g in range(nchunk):
            r0 = g * R
            pltpu.sync_copy(pred_hbm.at[wid, pl.ds(r0, R), :], pbuf)
            pltpu.sync_copy(targ_hbm.at[wid, pl.ds(r0, R), :], tbuf)

            def it(i, a):
                row = lax.shift_right_logical(i, 5)
                col = pl.multiple_of(
                    lax.shift_left(lax.bitwise_and(i, 31), 4), 16)
                p = pbuf[row, pl.ds(col, L)]
                t = tbuf[row, pl.ds(col, L)]
                d = p - t
                d2 = d * d
                return tuple(
                    a[b] + jnp.where(t >= jnp.float32(b), d2, zero)
                    for b in range(NBINS)
                )

            accs = lax.fori_loop(0, R * (W // L), it, accs)

        for b in range(NBINS):
            ov[pl.ds(b * L, L)] = accs[b]
        pltpu.sync_copy(ov, out_hbm.at[pl.ds(wid * NBINS * L, NBINS * L)])

    return body(pred, targ)


def kernel(prediction, target, weights):
    sc = _sc_fast(prediction, target, K_SC).reshape(NW, 2, L)
    tc_s, tc_mx, tc_mn = _tc_fast(prediction, target, K_SC, B - K_SC)

    s_fast = sc[:, 0, :].sum() + tc_s.sum()
    sc_clean = (lax.bitcast_convert_type(sc[:, 1, :], jnp.uint32).max()
                < jnp.uint32(ONE_BITS))
    tc_clean = jnp.logical_and(tc_mn.min() >= 0, tc_mx.max() < ONE_BITS)
    clean = jnp.logical_and(sc_clean, tc_clean)

    def fast_fn(_):
        return jnp.sqrt(s_fast * weights[0] / N)

    def general_fn(_):
        partials = _sc_general(prediction, target)
        a = partials.reshape(NW, NBINS, L).sum(axis=(0, 2))
        c = weights - jnp.concatenate(
            [jnp.zeros((1,), weights.dtype), weights[:-1]])
        return jnp.sqrt(jnp.dot(a, c) / N)

    return lax.cond(clean, fast_fn, general_fn, None)


# speedup vs baseline: 1.9811x; 1.9811x over previous
import functools

import jax
import jax.numpy as jnp
from jax import lax
from jax.experimental import pallas as pl
from jax.experimental.pallas import tpu as pltpu
from jax.experimental.pallas import tpu_sc as plsc

NC = 2
NS = 16
L = 16
NW = NC * NS

B = 32
H = 512
W = 512
N = B * H * W
NBINS = 5
U = 8
IPC = 1
NBUF = 4

ONE_BITS = 0x3F800000

K_SC = 8

_MESH = dict(core_axis_name="c", subcore_axis_name="s")


def _sc_fast(pred, targ, k):
    rows_w = k * H // NW
    r_chunk = min(32, rows_w)
    nchunk = rows_w // r_chunk
    blocks = r_chunk * (W // (L * U))

    @functools.partial(
        pl.kernel,
        out_type=jax.ShapeDtypeStruct((NW * 2 * L,), jnp.float32),
        mesh=plsc.VectorSubcoreMesh(**_MESH),
        scratch_types=[
            pltpu.VMEM((r_chunk, W), jnp.float32),
            pltpu.VMEM((r_chunk, W), jnp.float32),
            pltpu.VMEM((r_chunk, W), jnp.float32),
            pltpu.VMEM((r_chunk, W), jnp.float32),
            pltpu.VMEM((2 * L,), jnp.float32),
            pltpu.SemaphoreType.DMA,
            pltpu.SemaphoreType.DMA,
        ],
    )
    def body(pred_hbm, targ_hbm, out_hbm, pb0, tb0, pb1, tb1, ov, sm0, sm1):
        wid = lax.axis_index("s") * NC + lax.axis_index("c")
        r_abs = wid * rows_w
        img = lax.shift_right_logical(r_abs, 9)
        r_base = lax.bitwise_and(r_abs, H - 1)
        bufs = [(pb0, tb0, sm0), (pb1, tb1, sm1)]

        def start(g):
            pb, tb, sm = bufs[g % 2]
            r0 = pl.multiple_of(r_base + g * r_chunk, r_chunk)
            hp = pltpu.async_copy(pred_hbm.at[img, pl.ds(r0, r_chunk), :],
                                  pb, sm)
            ht = pltpu.async_copy(targ_hbm.at[img, pl.ds(r0, r_chunk), :],
                                  tb, sm)
            return (hp, ht)

        zero = jnp.zeros((L,), jnp.float32)
        uzero = jnp.zeros((L,), jnp.uint32)

        def make_it(pb, tb):
            def it(i, c):
                accs = list(c[:U])
                mx0, mx1 = c[U], c[U + 1]
                row = lax.shift_right_logical(i, 2)
                cb = pl.multiple_of(
                    lax.shift_left(lax.bitwise_and(i, 3), 7), 128)
                for j in range(U):
                    p = pb[row, pl.ds(cb + j * L, L)]
                    t = tb[row, pl.ds(cb + j * L, L)]
                    d = p - t
                    accs[j] = accs[j] + d * d
                    bu = lax.bitcast_convert_type(t, jnp.uint32)
                    if j % 2 == 0:
                        mx0 = jnp.where(bu > mx0, bu, mx0)
                    else:
                        mx1 = jnp.where(bu > mx1, bu, mx1)
                return (*accs, mx0, mx1)
            return it

        pending = {0: start(0)}
        carry = tuple(zero for _ in range(U)) + (uzero, uzero)
        for g in range(nchunk):
            if g + 1 < nchunk:
                pending[g + 1] = start(g + 1)
            for h in pending.pop(g):
                h.wait()
            pb, tb, _ = bufs[g % 2]
            carry = lax.fori_loop(0, blocks, make_it(pb, tb), carry)

        acc = carry[0]
        for j in range(1, U):
            acc = acc + carry[j]
        mx0, mx1 = carry[U], carry[U + 1]
        mx = jnp.where(mx0 > mx1, mx0, mx1)
        ov[pl.ds(0, L)] = acc
        ov[pl.ds(L, L)] = lax.bitcast_convert_type(mx, jnp.float32)
        pltpu.sync_copy(ov, out_hbm.at[pl.ds(wid * 2 * L, 2 * L)])

    return body(pred, targ)


def _tc_fast(pred, targ, k0, nimg):
    nstep = nimg // IPC
    assert nstep % NBUF == 0

    def body(p_hbm, t_hbm, s_out, mx_out, mn_out,
             pb, tb, acc_s, acc_mx, acc_mn, sems):
        i = pl.program_id(0)

        CROWS = IPC * H

        def copies(step, slot):
            r0 = pl.multiple_of((k0 * H) + step * CROWS, CROWS)
            cp = pltpu.make_async_copy(
                p_hbm.at[pl.ds(r0, CROWS)], pb.at[slot],
                sems.at[slot, 0])
            ct = pltpu.make_async_copy(
                t_hbm.at[pl.ds(r0, CROWS)], tb.at[slot],
                sems.at[slot, 1])
            return cp, ct

        def start(step, slot):
            cp, ct = copies(step, slot)
            cp.start()
            ct.start()

        @pl.when(i == 0)
        def _():
            acc_s[...] = jnp.zeros_like(acc_s)
            acc_mx[...] = jnp.full_like(acc_mx, jnp.iinfo(jnp.int32).min)
            acc_mn[...] = jnp.full_like(acc_mn, jnp.iinfo(jnp.int32).max)
            for k in range(NBUF - 1):
                start(k, k)

        base = i * NBUF
        for k in range(NBUF):
            step = base + k
            nxt = step + (NBUF - 1)
            nxt_slot = (k + NBUF - 1) % NBUF

            @pl.when(nxt < nstep)
            def _(nxt=nxt, nxt_slot=nxt_slot):
                start(nxt, nxt_slot)

            cp, ct = copies(step, k)
            cp.wait()
            ct.wait()

            for b in range(IPC):
                p = pb[k, pl.ds(b * H, H), :]
                t = tb[k, pl.ds(b * H, H), :]
                d = p - t
                d2 = d * d
                bi = lax.bitcast_convert_type(t, jnp.int32)
                for c in range(W // 128):
                    cs = slice(c * 128, (c + 1) * 128)
                    part = acc_s[:, cs]
                    mx = acc_mx[:, cs]
                    mn = acc_mn[:, cs]
                    for r in range(H // 8):
                        rs = slice(r * 8, (r + 1) * 8)
                        part = part + d2[rs, cs]
                        tile = bi[rs, cs]
                        mx = jnp.maximum(mx, tile)
                        mn = jnp.minimum(mn, tile)
                    acc_s[:, cs] = part
                    acc_mx[:, cs] = mx
                    acc_mn[:, cs] = mn

        @pl.when(i == nstep // NBUF - 1)
        def _():
            s_out[...] = acc_s[...]
            mx_out[...] = acc_mx[...]
            mn_out[...] = acc_mn[...]

    s, mx, mn = pl.pallas_call(
        body,
        grid=(nstep // NBUF,),
        in_specs=[
            pl.BlockSpec(memory_space=pl.ANY),
            pl.BlockSpec(memory_space=pl.ANY),
        ],
        out_specs=[
            pl.BlockSpec((8, W), lambda i: (0, 0)),
            pl.BlockSpec((8, W), lambda i: (0, 0)),
            pl.BlockSpec((8, W), lambda i: (0, 0)),
        ],
        out_shape=[
            jax.ShapeDtypeStruct((8, W), jnp.float32),
            jax.ShapeDtypeStruct((8, W), jnp.int32),
            jax.ShapeDtypeStruct((8, W), jnp.int32),
        ],
        scratch_shapes=[
            pltpu.VMEM((NBUF, IPC * H, W), jnp.float32),
            pltpu.VMEM((NBUF, IPC * H, W), jnp.float32),
            pltpu.VMEM((8, W), jnp.float32),
            pltpu.VMEM((8, W), jnp.int32),
            pltpu.VMEM((8, W), jnp.int32),
            pltpu.SemaphoreType.DMA((NBUF, 2)),
        ],
    )(pred.reshape(B * H, W), targ.reshape(B * H, W))
    return s, mx, mn


def _sc_general(pred, targ):
    R = 32
    nchunk = H // R

    @functools.partial(
        pl.kernel,
        out_type=jax.ShapeDtypeStruct((NW * NBINS * L,), jnp.float32),
        mesh=plsc.VectorSubcoreMesh(**_MESH),
        scratch_types=[
            pltpu.VMEM((R, W), jnp.float32),
            pltpu.VMEM((R, W), jnp.float32),
            pltpu.VMEM((NBINS * L,), jnp.float32),
        ],
    )
    def body(pred_hbm, targ_hbm, out_hbm, pbuf, tbuf, ov):
        wid = lax.axis_index("s") * NC + lax.axis_index("c")

        zero = jnp.zeros((L,), jnp.float32)
        accs = tuple(zero for _ in range(NBINS))
        for g in range(nchunk):
            r0 = g * R
            pltpu.sync_copy(pred_hbm.at[wid, pl.ds(r0, R), :], pbuf)
            pltpu.sync_copy(targ_hbm.at[wid, pl.ds(r0, R), :], tbuf)

            def it(i, a):
                row = lax.shift_right_logical(i, 5)
                col = pl.multiple_of(
                    lax.shift_left(lax.bitwise_and(i, 31), 4), 16)
                p = pbuf[row, pl.ds(col, L)]
                t = tbuf[row, pl.ds(col, L)]
                d = p - t
                d2 = d * d
                return tuple(
                    a[b] + jnp.where(t >= jnp.float32(b), d2, zero)
                    for b in range(NBINS)
                )

            accs = lax.fori_loop(0, R * (W // L), it, accs)

        for b in range(NBINS):
            ov[pl.ds(b * L, L)] = accs[b]
        pltpu.sync_copy(ov, out_hbm.at[pl.ds(wid * NBINS * L, NBINS * L)])

    return body(pred, targ)


def kernel(prediction, target, weights):
    sc = _sc_fast(prediction, target, K_SC).reshape(NW, 2, L)
    tc_s, tc_mx, tc_mn = _tc_fast(prediction, target, K_SC, B - K_SC)

    s_fast = sc[:, 0, :].sum() + tc_s.sum()
    sc_clean = (lax.bitcast_convert_type(sc[:, 1, :], jnp.uint32).max()
                < jnp.uint32(ONE_BITS))
    tc_clean = jnp.logical_and(tc_mn.min() >= 0, tc_mx.max() < ONE_BITS)
    clean = jnp.logical_and(sc_clean, tc_clean)

    def fast_fn(_):
        return jnp.sqrt(s_fast * weights[0] / N)

    def general_fn(_):
        partials = _sc_general(prediction, target)
        a = partials.reshape(NW, NBINS, L).sum(axis=(0, 2))
        c = weights - jnp.concatenate(
            [jnp.zeros((1,), weights.dtype), weights[:-1]])
        return jnp.sqrt(jnp.dot(a, c) / N)

    return lax.cond(clean, fast_fn, general_fn, None)
